# Initial kernel scaffold; baseline (speedup 1.0000x reference)
import jax
import jax.numpy as jnp
from jax.experimental import pallas as pl


def kernel(x, edge_index, batch, l0_W1, l0_b1, l0_W2, l0_b2, l0_g, l0_be, l1_W1, l1_b1, l1_W2, l1_b2, l1_g, l1_be, l2_W1, l2_b1, l2_W2, l2_b2, l2_g, l2_be):
    raise NotImplementedError("write your pallas kernel here")


# speedup vs baseline: 1.7939x; 1.7939x over previous
import functools

import jax
import jax.numpy as jnp
from jax import lax
from jax.experimental import pallas as pl
from jax.experimental.pallas import tpu as pltpu
from jax.experimental.pallas import tpu_sc as plsc

N = 10000
E = 320000
F_IN = 128
H = 512
BN_EPS = 1e-5

NCORES = 2
NSUB = 16
NWORK = NCORES * NSUB
EB = 128
RPT = 80
EPW = RPT * EB
GR = 16
NGRP = RPT // GR
ZB = 32
TRASH = N
ACC_ROWS = NSUB * 640


def _sc_segsum(h2, src3, dst3, C):
    mesh = plsc.VectorSubcoreMesh(core_axis_name="c", subcore_axis_name="s")
    out_type = jax.ShapeDtypeStruct((NCORES, C, ACC_ROWS, 128), jnp.float32)

    @functools.partial(
        pl.kernel,
        out_type=out_type,
        mesh=mesh,
        scratch_types=[
            pltpu.VMEM_SHARED((ACC_ROWS, 128), jnp.float32),
            pltpu.VMEM((GR, EB), jnp.int32),
            pltpu.VMEM((GR, EB), jnp.int32),
            pltpu.VMEM((EB, 128), jnp.float32),
            pltpu.VMEM((EB, 128), jnp.float32),
            pltpu.VMEM((ZB, 128), jnp.float32),
            pltpu.SemaphoreType.DMA,
            pltpu.SemaphoreType.DMA,
        ],
    )
    def seg_kernel(h2_hbm, src_hbm, dst_hbm, out_hbm,
                   acc, idx_v, dst_v, rb0, rb1, zob, sem0, sem1):
        cid = lax.axis_index("c")
        sid = lax.axis_index("s")
        wid = cid * NSUB + sid

        for ck in range(C):
            @pl.loop(0, ZB)
            def _zrow(i):
                for b in range(128 // 16):
                    zob[i, pl.ds(b * 16, 16)] = jnp.zeros((16,), jnp.float32)

            for z in range(20):
                pltpu.sync_copy(zob, acc.at[pl.ds(sid * 640 + z * ZB, ZB)])
            plsc.subcore_barrier()

            for g in range(NGRP):
                pltpu.sync_copy(src_hbm.at[wid, pl.ds(g * GR, GR)], idx_v)
                pltpu.sync_copy(dst_hbm.at[wid, pl.ds(g * GR, GR)], dst_v)
                if C > 1:
                    @pl.loop(0, GR)
                    def _gidx(j):
                        for b in range(EB // 16):
                            sl = pl.ds(b * 16, 16)
                            idx_v[j, sl] = idx_v[j, sl] * C + ck

                pltpu.async_copy(h2_hbm.at[idx_v.at[0]], rb0, sem0)

                @pl.loop(0, GR // 2 - 1)
                def _pair(t):
                    j = 2 * t
                    pltpu.async_copy(h2_hbm.at[idx_v.at[j + 1]], rb1, sem1)
                    pltpu.make_async_copy(h2_hbm.at[idx_v.at[j]], rb0, sem0).wait()
                    pltpu.sync_copy(rb0, acc.at[dst_v.at[j]], add=True)
                    pltpu.async_copy(h2_hbm.at[idx_v.at[j + 2]], rb0, sem0)
                    pltpu.make_async_copy(h2_hbm.at[idx_v.at[j + 1]], rb1, sem1).wait()
                    pltpu.sync_copy(rb1, acc.at[dst_v.at[j + 1]], add=True)

                pltpu.async_copy(h2_hbm.at[idx_v.at[GR - 1]], rb1, sem1)
                pltpu.make_async_copy(h2_hbm.at[idx_v.at[GR - 2]], rb0, sem0).wait()
                pltpu.sync_copy(rb0, acc.at[dst_v.at[GR - 2]], add=True)
                pltpu.make_async_copy(h2_hbm.at[idx_v.at[GR - 1]], rb1, sem1).wait()
                pltpu.sync_copy(rb1, acc.at[dst_v.at[GR - 1]], add=True)

            plsc.subcore_barrier()

            for z in range(20):
                r0 = sid * 640 + z * ZB
                pltpu.sync_copy(acc.at[pl.ds(r0, ZB)], zob)
                pltpu.sync_copy(zob, out_hbm.at[cid, ck, pl.ds(r0, ZB)])
            plsc.subcore_barrier()

    return seg_kernel(h2, src3, dst3)


def _tc_mlp(h, aggp, W1, b1, W2, b2, bn=1000):
    Nn, F = h.shape
    C = F // 128
    ng = Nn // bn

    def body(h_ref, ag_ref, w1_ref, b1_ref, w2_ref, b2_ref, a2_ref, st_ref):
        i = pl.program_id(0)
        z = jnp.concatenate(
            [h_ref[:, c * 128:(c + 1) * 128] + (ag_ref[0, c] + ag_ref[1, c])
             for c in range(C)], axis=1)
        acc = jnp.dot(z, w1_ref[...], preferred_element_type=jnp.float32)
        a1 = jnp.maximum(acc + b1_ref[...], 0.0)
        a2 = jnp.dot(a1, w2_ref[...],
                     preferred_element_type=jnp.float32) + b2_ref[...]
        a2 = jnp.maximum(a2, 0.0)
        a2_ref[...] = a2

        @pl.when(i == 0)
        def _():
            st_ref[...] = jnp.zeros_like(st_ref)

        st_ref[0:1, :] += jnp.sum(a2, axis=0, keepdims=True)

    return pl.pallas_call(
        body,
        grid=(ng,),
        in_specs=[
            pl.BlockSpec((bn, F), lambda i: (i, 0)),
            pl.BlockSpec((NCORES, C, bn, 128), lambda i: (0, 0, i, 0)),
            pl.BlockSpec((F, H), lambda i: (0, 0)),
            pl.BlockSpec((1, H), lambda i: (0, 0)),
            pl.BlockSpec((H, H), lambda i: (0, 0)),
            pl.BlockSpec((1, H), lambda i: (0, 0)),
        ],
        out_specs=[
            pl.BlockSpec((bn, H), lambda i: (i, 0)),
            pl.BlockSpec((8, H), lambda i: (0, 0)),
        ],
        out_shape=[
            jax.ShapeDtypeStruct((Nn, H), jnp.float32),
            jax.ShapeDtypeStruct((8, H), jnp.float32),
        ],
    )(h, aggp, W1, b1, W2, b2)


def _tc_post(a2, st, g, be, res, mode, bn=1000):
    Nn = a2.shape[0]
    ng = Nn // bn

    def body(*refs):
        if mode == 1:
            a2_ref, st_ref, g_ref, be_ref, res_ref, o_ref, vacc = refs
        else:
            a2_ref, st_ref, g_ref, be_ref, o_ref, vacc = refs
        p = pl.program_id(0)
        i = pl.program_id(1)
        mean = st_ref[0:1, :] * (1.0 / N)

        @pl.when(jnp.logical_and(p == 0, i == 0))
        def _():
            vacc[...] = jnp.zeros_like(vacc)

        @pl.when(p == 0)
        def _():
            dlt = a2_ref[...] - mean
            vacc[0:1, :] += jnp.sum(dlt * dlt, axis=0, keepdims=True)

        @pl.when(p == 1)
        def _():
            var = vacc[0:1, :] * (1.0 / N)
            y = (a2_ref[...] - mean) / jnp.sqrt(var + BN_EPS) * g_ref[...] + be_ref[...]
            if mode == 0:
                y = jnp.maximum(y, 0.0)
            elif mode == 1:
                y = jnp.maximum(y + res_ref[...], 0.0)
            o_ref[...] = y

    in_specs = [
        pl.BlockSpec((bn, H), lambda p, i: (i, 0)),
        pl.BlockSpec((8, H), lambda p, i: (0, 0)),
        pl.BlockSpec((1, H), lambda p, i: (0, 0)),
        pl.BlockSpec((1, H), lambda p, i: (0, 0)),
    ]
    args = [a2, st, g, be]
    if mode == 1:
        in_specs.append(pl.BlockSpec((bn, H), lambda p, i: (i, 0)))
        args.append(res)

    return pl.pallas_call(
        body,
        grid=(2, ng),
        in_specs=in_specs,
        out_specs=pl.BlockSpec((bn, H), lambda p, i: (i, 0)),
        out_shape=jax.ShapeDtypeStruct((Nn, H), jnp.float32),
        scratch_shapes=[pltpu.VMEM((8, H), jnp.float32)],
    )(*args)


def kernel(x, edge_index, batch,
           l0_W1, l0_b1, l0_W2, l0_b2, l0_g, l0_be,
           l1_W1, l1_b1, l1_W2, l1_b2, l1_g, l1_be,
           l2_W1, l2_b1, l2_W2, l2_b2, l2_g, l2_be):
    src = edge_index[0]
    dst = edge_index[1]
    pad = NWORK * EPW - E
    src3 = jnp.concatenate([src, jnp.zeros((pad,), jnp.int32)]).reshape(NWORK, RPT, EB)
    dst3 = jnp.concatenate([dst, jnp.full((pad,), TRASH, jnp.int32)]).reshape(NWORK, RPT, EB)

    params = [
        (l0_W1, l0_b1, l0_W2, l0_b2, l0_g, l0_be),
        (l1_W1, l1_b1, l1_W2, l1_b2, l1_g, l1_be),
        (l2_W1, l2_b1, l2_W2, l2_b2, l2_g, l2_be),
    ]

    h = x
    for i, (W1, b1, W2, b2, g, be) in enumerate(params):
        F = h.shape[1]
        C = F // 128
        h2 = h.reshape(N * C, 128)
        aggp = _sc_segsum(h2, src3, dst3, C)
        a2, st = _tc_mlp(h, aggp, W1, b1.reshape(1, H), W2, b2.reshape(1, H))
        mode = 2 if i == 2 else (0 if i == 0 else 1)
        h = _tc_post(a2, st, g.reshape(1, H), be.reshape(1, H),
                     h if mode == 1 else None, mode)
    return h


# speedup vs baseline: 1.8161x; 1.0124x over previous
import functools

import jax
import jax.numpy as jnp
from jax import lax
from jax.experimental import pallas as pl
from jax.experimental.pallas import tpu as pltpu
from jax.experimental.pallas import tpu_sc as plsc

N = 10000
E = 320000
F_IN = 128
H = 512
BN_EPS = 1e-5

NCORES = 2
NSUB = 16
NWORK = NCORES * NSUB
EB = 128
RPT = 80
EPW = RPT * EB
GR = 16
NGRP = RPT // GR
ZB = 32
TRASH = N
ACC_ROWS = NSUB * 640


def _sc_segsum(h2, src3, dst3, C):
    mesh = plsc.VectorSubcoreMesh(core_axis_name="c", subcore_axis_name="s")
    out_type = jax.ShapeDtypeStruct((NCORES, C, ACC_ROWS, 128), jnp.float32)

    @functools.partial(
        pl.kernel,
        out_type=out_type,
        mesh=mesh,
        scratch_types=[
            pltpu.VMEM_SHARED((ACC_ROWS, 128), jnp.float32),
            pltpu.VMEM((GR, EB), jnp.int32),
            pltpu.VMEM((GR, EB), jnp.int32),
            pltpu.VMEM((EB, 128), jnp.float32),
            pltpu.VMEM((EB, 128), jnp.float32),
            pltpu.VMEM((ZB, 128), jnp.float32),
            pltpu.SemaphoreType.DMA,
            pltpu.SemaphoreType.DMA,
        ],
    )
    def seg_kernel(h2_hbm, src_hbm, dst_hbm, out_hbm,
                   acc, idx_v, dst_v, rb0, rb1, zob, sem0, sem1):
        cid = lax.axis_index("c")
        sid = lax.axis_index("s")
        wid = cid * NSUB + sid

        for ck in range(C):
            @pl.loop(0, ZB)
            def _zrow(i):
                for b in range(128 // 16):
                    zob[i, pl.ds(b * 16, 16)] = jnp.zeros((16,), jnp.float32)

            for z in range(20):
                pltpu.sync_copy(zob, acc.at[pl.ds(sid * 640 + z * ZB, ZB)])
            plsc.subcore_barrier()

            for g in range(NGRP):
                pltpu.sync_copy(src_hbm.at[wid, pl.ds(g * GR, GR)], idx_v)
                pltpu.sync_copy(dst_hbm.at[wid, pl.ds(g * GR, GR)], dst_v)
                if C > 1:
                    @pl.loop(0, GR)
                    def _gidx(j):
                        for b in range(EB // 16):
                            sl = pl.ds(b * 16, 16)
                            idx_v[j, sl] = idx_v[j, sl] * C + ck

                pltpu.async_copy(h2_hbm.at[idx_v.at[0]], rb0, sem0)

                @pl.loop(0, GR // 2 - 1)
                def _pair(t):
                    j = 2 * t
                    pltpu.async_copy(h2_hbm.at[idx_v.at[j + 1]], rb1, sem1)
                    pltpu.make_async_copy(h2_hbm.at[idx_v.at[j]], rb0, sem0).wait()
                    pltpu.sync_copy(rb0, acc.at[dst_v.at[j]], add=True)
                    pltpu.async_copy(h2_hbm.at[idx_v.at[j + 2]], rb0, sem0)
                    pltpu.make_async_copy(h2_hbm.at[idx_v.at[j + 1]], rb1, sem1).wait()
                    pltpu.sync_copy(rb1, acc.at[dst_v.at[j + 1]], add=True)

                pltpu.async_copy(h2_hbm.at[idx_v.at[GR - 1]], rb1, sem1)
                pltpu.make_async_copy(h2_hbm.at[idx_v.at[GR - 2]], rb0, sem0).wait()
                pltpu.sync_copy(rb0, acc.at[dst_v.at[GR - 2]], add=True)
                pltpu.make_async_copy(h2_hbm.at[idx_v.at[GR - 1]], rb1, sem1).wait()
                pltpu.sync_copy(rb1, acc.at[dst_v.at[GR - 1]], add=True)

            plsc.subcore_barrier()

            for z in range(20):
                r0 = sid * 640 + z * ZB
                pltpu.sync_copy(acc.at[pl.ds(r0, ZB)], zob)
                pltpu.sync_copy(zob, out_hbm.at[cid, ck, pl.ds(r0, ZB)])
            plsc.subcore_barrier()

    return seg_kernel(h2, src3, dst3)


def _tc_mlp(h, aggp, W1, b1, W2, b2, bn=1000):
    Nn, F = h.shape
    C = F // 128
    ng = Nn // bn

    def body(h_ref, ag_ref, w1_ref, b1_ref, w2_ref, b2_ref, a2_ref, st_ref):
        i = pl.program_id(0)
        z = jnp.concatenate(
            [h_ref[:, c * 128:(c + 1) * 128] + (ag_ref[0, c] + ag_ref[1, c])
             for c in range(C)], axis=1)
        acc = jnp.dot(z, w1_ref[...], preferred_element_type=jnp.float32)
        a1 = jnp.maximum(acc + b1_ref[...], 0.0)
        a2 = jnp.dot(a1, w2_ref[...],
                     preferred_element_type=jnp.float32) + b2_ref[...]
        a2 = jnp.maximum(a2, 0.0)
        a2_ref[...] = a2

        @pl.when(i == 0)
        def _():
            st_ref[...] = jnp.zeros_like(st_ref)

        st_ref[...] += jnp.sum(a2.reshape(bn // 8, 8, H), axis=0)

    return pl.pallas_call(
        body,
        grid=(ng,),
        in_specs=[
            pl.BlockSpec((bn, F), lambda i: (i, 0)),
            pl.BlockSpec((NCORES, C, bn, 128), lambda i: (0, 0, i, 0)),
            pl.BlockSpec((F, H), lambda i: (0, 0)),
            pl.BlockSpec((1, H), lambda i: (0, 0)),
            pl.BlockSpec((H, H), lambda i: (0, 0)),
            pl.BlockSpec((1, H), lambda i: (0, 0)),
        ],
        out_specs=[
            pl.BlockSpec((bn, H), lambda i: (i, 0)),
            pl.BlockSpec((8, H), lambda i: (0, 0)),
        ],
        out_shape=[
            jax.ShapeDtypeStruct((Nn, H), jnp.float32),
            jax.ShapeDtypeStruct((8, H), jnp.float32),
        ],
    )(h, aggp, W1, b1, W2, b2)


def _tc_post(a2, st, g, be, res, mode, bn=1000):
    Nn = a2.shape[0]
    ng = Nn // bn

    def _fold8(v):
        a = v[0:1, :] + v[4:5, :]
        b = v[1:2, :] + v[5:6, :]
        c = v[2:3, :] + v[6:7, :]
        d = v[3:4, :] + v[7:8, :]
        return (a + c) + (b + d)

    def body(*refs):
        if mode == 1:
            a2_ref, st_ref, g_ref, be_ref, res_ref, o_ref, vacc = refs
        else:
            a2_ref, st_ref, g_ref, be_ref, o_ref, vacc = refs
        p = pl.program_id(0)
        i = pl.program_id(1)
        mean = _fold8(st_ref[...]) / float(N)

        @pl.when(jnp.logical_and(p == 0, i == 0))
        def _():
            vacc[...] = jnp.zeros_like(vacc)

        @pl.when(p == 0)
        def _():
            dlt = a2_ref[...] - mean
            vacc[...] += jnp.sum((dlt * dlt).reshape(bn // 8, 8, H), axis=0)

        @pl.when(p == 1)
        def _():
            var = _fold8(vacc[...]) / float(N)
            y = (a2_ref[...] - mean) / jnp.sqrt(var + BN_EPS) * g_ref[...] + be_ref[...]
            if mode == 0:
                y = jnp.maximum(y, 0.0)
            elif mode == 1:
                y = jnp.maximum(y + res_ref[...], 0.0)
            o_ref[...] = y

    in_specs = [
        pl.BlockSpec((bn, H), lambda p, i: (i, 0)),
        pl.BlockSpec((8, H), lambda p, i: (0, 0)),
        pl.BlockSpec((1, H), lambda p, i: (0, 0)),
        pl.BlockSpec((1, H), lambda p, i: (0, 0)),
    ]
    args = [a2, st, g, be]
    if mode == 1:
        in_specs.append(pl.BlockSpec((bn, H), lambda p, i: (i, 0)))
        args.append(res)

    return pl.pallas_call(
        body,
        grid=(2, ng),
        in_specs=in_specs,
        out_specs=pl.BlockSpec((bn, H), lambda p, i: (i, 0)),
        out_shape=jax.ShapeDtypeStruct((Nn, H), jnp.float32),
        scratch_shapes=[pltpu.VMEM((8, H), jnp.float32)],
    )(*args)


def kernel(x, edge_index, batch,
           l0_W1, l0_b1, l0_W2, l0_b2, l0_g, l0_be,
           l1_W1, l1_b1, l1_W2, l1_b2, l1_g, l1_be,
           l2_W1, l2_b1, l2_W2, l2_b2, l2_g, l2_be):
    src = edge_index[0]
    dst = edge_index[1]
    pad = NWORK * EPW - E
    src3 = jnp.concatenate([src, jnp.zeros((pad,), jnp.int32)]).reshape(NWORK, RPT, EB)
    dst3 = jnp.concatenate([dst, jnp.full((pad,), TRASH, jnp.int32)]).reshape(NWORK, RPT, EB)

    params = [
        (l0_W1, l0_b1, l0_W2, l0_b2, l0_g, l0_be),
        (l1_W1, l1_b1, l1_W2, l1_b2, l1_g, l1_be),
        (l2_W1, l2_b1, l2_W2, l2_b2, l2_g, l2_be),
    ]

    h = x
    for i, (W1, b1, W2, b2, g, be) in enumerate(params):
        F = h.shape[1]
        C = F // 128
        h2 = h.reshape(N * C, 128)
        aggp = _sc_segsum(h2, src3, dst3, C)
        a2, st = _tc_mlp(h, aggp, W1, b1.reshape(1, H), W2, b2.reshape(1, H))
        mode = 2 if i == 2 else (0 if i == 0 else 1)
        h = _tc_post(a2, st, g.reshape(1, H), be.reshape(1, H),
                     h if mode == 1 else None, mode)
    return h
